# baseline (speedup 1.0000x reference)
import functools

import jax
import jax.numpy as jnp
import numpy as np
from jax.experimental import pallas as pl
from jax.experimental.pallas import tpu as pltpu

_COMPUTE_DTYPE = jnp.bfloat16


def _fused_block_kernel(x_ref, w1_ref, b1_ref, wd_ref, bd_ref,
                        wcat_ref, b3_ref, mwl_ref, mwr_ref, mhu_ref,
                        mhd_ref, even_ref, out_ref, *, W):
    C, HW = x_ref.shape
    Cb = C // 2

    xb = x_ref[...].astype(_COMPUTE_DTYPE)
    x1 = xb[:Cb]
    x2 = xb[Cb:]

    t = jnp.dot(w1_ref[...], x2, preferred_element_type=jnp.float32)
    t = jnp.maximum(t + b1_ref[...], 0.0)

    wd = wd_ref[...]
    tl = pltpu.roll(t, 1, 1) * mwl_ref[...]
    tr = pltpu.roll(t, HW - 1, 1) * mwr_ref[...]
    u0 = tl * wd[:, 0:1] + t * wd[:, 1:2] + tr * wd[:, 2:3]
    u1 = tl * wd[:, 3:4] + t * wd[:, 4:5] + tr * wd[:, 5:6]
    u2 = tl * wd[:, 6:7] + t * wd[:, 7:8] + tr * wd[:, 8:9]
    d = (pltpu.roll(u0, W, 1) * mhu_ref[...] + u1
         + pltpu.roll(u2, HW - W, 1) * mhd_ref[...] + bd_ref[...])

    v = jnp.concatenate([d.astype(_COMPUTE_DTYPE), x1], axis=0)
    z = (jnp.dot(wcat_ref[...], v, preferred_element_type=jnp.float32)
         + b3_ref[...])
    out_ref[...] = jnp.maximum(z, z * even_ref[...]).astype(out_ref.dtype)


def _fold(params):
    w1, s1, b1, wdw, s2, b2, w3, s3, b3 = params
    Cb = w1.shape[0]
    C = 2 * Cb
    w1f = (w1 * s1[:, None]).astype(_COMPUTE_DTYPE)
    b1c = b1.reshape(Cb, 1).astype(jnp.float32)
    wdf = (wdw * s2[:, None, None]).reshape(Cb, 9).astype(jnp.float32)
    bdc = b2.reshape(Cb, 1).astype(jnp.float32)
    wcat = jnp.zeros((C, 2 * Cb), jnp.float32)
    wcat = wcat.at[1::2, :Cb].set(w3 * s3[:, None])
    wcat = wcat.at[0::2, Cb:].set(jnp.eye(Cb, dtype=jnp.float32))
    wcat = wcat.astype(_COMPUTE_DTYPE)
    b3c = jnp.zeros((C, 1), jnp.float32).at[1::2, 0].set(b3)
    evenc = jnp.zeros((C, 1), jnp.float32).at[0::2, 0].set(1.0)
    return w1f, b1c, wdf, bdc, wcat, b3c, evenc


@jax.jit
def kernel(x, w1, s1, b1, wdw, s2, b2, w3, s3, b3):
    N, C, H, W = x.shape
    HW = H * W
    Cb = C // 2

    w1f, b1c, wdf, bdc, wcat, b3c, evenc = _fold(
        (w1, s1, b1, wdw, s2, b2, w3, s3, b3))

    col = jnp.arange(HW, dtype=jnp.int32) % W
    row = jnp.arange(HW, dtype=jnp.int32) // W
    mwl = (col > 0).astype(jnp.float32).reshape(1, HW)
    mwr = (col < W - 1).astype(jnp.float32).reshape(1, HW)
    mhu = (row > 0).astype(jnp.float32).reshape(1, HW)
    mhd = (row < H - 1).astype(jnp.float32).reshape(1, HW)

    x3 = x.reshape(N, C, HW)

    kernel_fn = functools.partial(_fused_block_kernel, W=W)
    const = lambda a: pl.BlockSpec(a.shape, lambda n: (0,) * a.ndim)

    flops = int(N * (2 * Cb * Cb * HW + 2 * C * 2 * Cb * HW + 24 * Cb * HW))
    bytes_accessed = int(4 * N * C * HW + 2 * N * C * HW
                         + (w1f.size + wcat.size) * 2
                         + (b1c.size + wdf.size + bdc.size + b3c.size
                            + evenc.size + 4 * HW) * 4)

    out3 = pl.pallas_call(
        kernel_fn,
        out_shape=jax.ShapeDtypeStruct((N, C, HW), _COMPUTE_DTYPE),
        grid_spec=pltpu.PrefetchScalarGridSpec(
            num_scalar_prefetch=0,
            grid=(N,),
            in_specs=[
                pl.BlockSpec((pl.Squeezed(), C, HW), lambda n: (n, 0, 0)),
                const(w1f), const(b1c), const(wdf), const(bdc),
                const(wcat), const(b3c),
                const(mwl), const(mwr), const(mhu), const(mhd),
                const(evenc),
            ],
            out_specs=pl.BlockSpec((pl.Squeezed(), C, HW), lambda n: (n, 0, 0)),
        ),
        compiler_params=pltpu.CompilerParams(
            dimension_semantics=("parallel",)),
        cost_estimate=pl.CostEstimate(flops=flops, transcendentals=0,
                                      bytes_accessed=bytes_accessed),
    )(x3, w1f, b1c, wdf, bdc, wcat, b3c, mwl, mwr, mhu, mhd, evenc)
    return out3.reshape(N, C, H, W)


# speedup vs baseline: 1.1966x; 1.1966x over previous
import functools
import math

import jax
import jax.numpy as jnp
import numpy as np
from jax.experimental import pallas as pl
from jax.experimental.pallas import tpu as pltpu

_COMPUTE_DTYPE = jnp.bfloat16


def _pixmajor_kernel(x_ref, w1t_ref, b1_ref, wd_ref, bd_ref,
                     wcatt_ref, b3_ref, even_ref, out_ref, *, H, W):
    HW, NB, C = x_ref.shape
    Cb = C // 2
    M = HW * NB

    xb = x_ref[...].astype(_COMPUTE_DTYPE)
    x1 = xb[:, :, :Cb].reshape(M, Cb)
    x2 = xb[:, :, Cb:].reshape(M, Cb)

    t = jnp.dot(x2, w1t_ref[...], preferred_element_type=jnp.float32)
    t = jnp.maximum(t + b1_ref[...], 0.0)

    t4 = t.reshape(H, W, NB, Cb)
    zw = jnp.zeros((H, 1, NB, Cb), jnp.float32)
    tw = jnp.concatenate([zw, t4, zw], axis=1)
    zh = jnp.zeros((1, W + 2, NB, Cb), jnp.float32)
    tp = jnp.concatenate([zh, tw, zh], axis=0)

    wd = wd_ref[...]
    d = None
    for a in range(3):
        for b in range(3):
            term = tp[a:a + H, b:b + W] * wd[3 * a + b].reshape(1, 1, 1, Cb)
            d = term if d is None else d + term
    d = (d + bd_ref[...].reshape(1, 1, 1, Cb)).reshape(M, Cb)

    vm = jnp.concatenate([d.astype(_COMPUTE_DTYPE), x1], axis=1)
    z = (jnp.dot(vm, wcatt_ref[...], preferred_element_type=jnp.float32)
         + b3_ref[...])
    z = jnp.maximum(z, z * even_ref[...])
    out_ref[...] = z.astype(out_ref.dtype).reshape(HW, NB, C)


def _fold(params):
    w1, s1, b1, wdw, s2, b2, w3, s3, b3 = params
    Cb = w1.shape[0]
    C = 2 * Cb
    w1t = (w1 * s1[:, None]).T.astype(_COMPUTE_DTYPE)
    b1c = b1.reshape(1, Cb).astype(jnp.float32)
    wdf = (wdw * s2[:, None, None]).reshape(Cb, 9).T.astype(jnp.float32)
    bdc = b2.reshape(1, Cb).astype(jnp.float32)
    wcat = jnp.zeros((C, 2 * Cb), jnp.float32)
    wcat = wcat.at[1::2, :Cb].set(w3 * s3[:, None])
    wcat = wcat.at[0::2, Cb:].set(jnp.eye(Cb, dtype=jnp.float32))
    wcatt = wcat.T.astype(_COMPUTE_DTYPE)
    b3c = jnp.zeros((1, C), jnp.float32).at[0, 1::2].set(b3)
    evenc = jnp.zeros((1, C), jnp.float32).at[0, 0::2].set(1.0)
    return w1t, b1c, wdf, bdc, wcatt, b3c, evenc


@jax.jit
def kernel(x, w1, s1, b1, wdw, s2, b2, w3, s3, b3):
    N, C, H, W = x.shape
    HW = H * W
    Cb = C // 2

    w1t, b1c, wdf, bdc, wcatt, b3c, evenc = _fold(
        (w1, s1, b1, wdw, s2, b2, w3, s3, b3))

    xt = jnp.transpose(x.reshape(N, C, HW), (2, 0, 1))

    NB = math.gcd(N, 8)
    kernel_fn = functools.partial(_pixmajor_kernel, H=H, W=W)
    const = lambda a: pl.BlockSpec(a.shape, lambda n: (0,) * a.ndim)

    flops = int(N * (2 * Cb * Cb * HW + 2 * C * 2 * Cb * HW + 24 * Cb * HW))
    bytes_accessed = int(6 * N * C * HW)

    outt = pl.pallas_call(
        kernel_fn,
        out_shape=jax.ShapeDtypeStruct((HW, N, C), _COMPUTE_DTYPE),
        grid_spec=pltpu.PrefetchScalarGridSpec(
            num_scalar_prefetch=0,
            grid=(N // NB,),
            in_specs=[
                pl.BlockSpec((HW, NB, C), lambda n: (0, n, 0)),
                const(w1t), const(b1c), const(wdf), const(bdc),
                const(wcatt), const(b3c), const(evenc),
            ],
            out_specs=pl.BlockSpec((HW, NB, C), lambda n: (0, n, 0)),
        ),
        compiler_params=pltpu.CompilerParams(
            dimension_semantics=("parallel",)),
        cost_estimate=pl.CostEstimate(flops=flops, transcendentals=0,
                                      bytes_accessed=bytes_accessed),
    )(xt, w1t, b1c, wdf, bdc, wcatt, b3c, evenc)
    return jnp.transpose(outt, (1, 2, 0)).reshape(N, C, H, W)


# speedup vs baseline: 1.1970x; 1.0004x over previous
import functools
import math

import jax
import jax.numpy as jnp
import numpy as np
from jax.experimental import pallas as pl
from jax.experimental.pallas import tpu as pltpu

_COMPUTE_DTYPE = jnp.bfloat16


def _pixmajor_kernel(x_ref, w1t_ref, b1_ref, wd_ref, bd_ref,
                     wcatt_ref, b3_ref, even_ref, out_ref, *, H, W):
    HW, NB, C = x_ref.shape
    Cb = C // 2
    M = HW * NB

    xb = x_ref[...].astype(_COMPUTE_DTYPE)
    x1 = xb[:, :, :Cb].reshape(M, Cb)
    x2 = xb[:, :, Cb:].reshape(M, Cb)

    t = jnp.dot(x2, w1t_ref[...], preferred_element_type=jnp.float32)
    t = jnp.maximum(t + b1_ref[...], 0.0)

    t4 = t.reshape(H, W, NB, Cb)
    zw = jnp.zeros((H, 1, NB, Cb), jnp.float32)
    tw = jnp.concatenate([zw, t4, zw], axis=1)
    zh = jnp.zeros((1, W + 2, NB, Cb), jnp.float32)
    tp = jnp.concatenate([zh, tw, zh], axis=0)

    wd = wd_ref[...]
    d = None
    for a in range(3):
        for b in range(3):
            term = tp[a:a + H, b:b + W] * wd[3 * a + b].reshape(1, 1, 1, Cb)
            d = term if d is None else d + term
    d = (d + bd_ref[...].reshape(1, 1, 1, Cb)).reshape(M, Cb)

    vm = jnp.concatenate([d.astype(_COMPUTE_DTYPE), x1], axis=1)
    z = (jnp.dot(vm, wcatt_ref[...], preferred_element_type=jnp.float32)
         + b3_ref[...])
    z = jnp.maximum(z, z * even_ref[...])
    out_ref[...] = z.astype(out_ref.dtype).reshape(HW, NB, C)


def _fold(params):
    w1, s1, b1, wdw, s2, b2, w3, s3, b3 = params
    Cb = w1.shape[0]
    C = 2 * Cb
    w1t = (w1 * s1[:, None]).T.astype(_COMPUTE_DTYPE)
    b1c = b1.reshape(1, Cb).astype(jnp.float32)
    wdf = (wdw * s2[:, None, None]).reshape(Cb, 9).T.astype(jnp.float32)
    bdc = b2.reshape(1, Cb).astype(jnp.float32)
    wcat = jnp.zeros((C, 2 * Cb), jnp.float32)
    wcat = wcat.at[1::2, :Cb].set(w3 * s3[:, None])
    wcat = wcat.at[0::2, Cb:].set(jnp.eye(Cb, dtype=jnp.float32))
    wcatt = wcat.T.astype(_COMPUTE_DTYPE)
    b3c = jnp.zeros((1, C), jnp.float32).at[0, 1::2].set(b3)
    evenc = jnp.zeros((1, C), jnp.float32).at[0, 0::2].set(1.0)
    return w1t, b1c, wdf, bdc, wcatt, b3c, evenc


@jax.jit
def kernel(x, w1, s1, b1, wdw, s2, b2, w3, s3, b3):
    N, C, H, W = x.shape
    HW = H * W
    Cb = C // 2

    w1t, b1c, wdf, bdc, wcatt, b3c, evenc = _fold(
        (w1, s1, b1, wdw, s2, b2, w3, s3, b3))

    xt = jnp.transpose(x, (2, 3, 0, 1)).reshape(HW, N, C)

    NB = math.gcd(N, 8)
    kernel_fn = functools.partial(_pixmajor_kernel, H=H, W=W)
    const = lambda a: pl.BlockSpec(a.shape, lambda n: (0,) * a.ndim)

    flops = int(N * (2 * Cb * Cb * HW + 2 * C * 2 * Cb * HW + 24 * Cb * HW))
    bytes_accessed = int(6 * N * C * HW)

    outt = pl.pallas_call(
        kernel_fn,
        out_shape=jax.ShapeDtypeStruct((HW, N, C), _COMPUTE_DTYPE),
        grid_spec=pltpu.PrefetchScalarGridSpec(
            num_scalar_prefetch=0,
            grid=(N // NB,),
            in_specs=[
                pl.BlockSpec((HW, NB, C), lambda n: (0, n, 0)),
                const(w1t), const(b1c), const(wdf), const(bdc),
                const(wcatt), const(b3c), const(evenc),
            ],
            out_specs=pl.BlockSpec((HW, NB, C), lambda n: (0, n, 0)),
        ),
        compiler_params=pltpu.CompilerParams(
            dimension_semantics=("parallel",)),
        cost_estimate=pl.CostEstimate(flops=flops, transcendentals=0,
                                      bytes_accessed=bytes_accessed),
    )(xt, w1t, b1c, wdf, bdc, wcatt, b3c, evenc)
    return jnp.transpose(outt.reshape(H, W, N, C), (2, 3, 0, 1))
